# Initial kernel scaffold; baseline (speedup 1.0000x reference)
import jax
import jax.numpy as jnp
from jax.experimental import pallas as pl


def kernel(input_seq, W0, W1, W2):
    raise NotImplementedError("write your pallas kernel here")


# speedup vs baseline: 1.6431x; 1.6431x over previous
import functools

import jax
import jax.numpy as jnp
from jax import lax
from jax.experimental import pallas as pl
from jax.experimental.pallas import tpu as pltpu
from jax.experimental.pallas import tpu_sc as plsc

NUM_LAYERS = 12
NUM_TABLES = 3


def _sc_lookup(idx, w0, w1, w2):
    (n,) = idx.shape
    v, d = w0.shape

    info = plsc.get_sparse_core_info()
    nc, ns = info.num_cores, info.num_subcores
    nw = nc * ns
    tpw = n // nw
    chunk = 32
    nchunk = tpw // chunk

    mesh = plsc.VectorSubcoreMesh(core_axis_name="c", subcore_axis_name="s")

    @functools.partial(
        pl.kernel,
        mesh=mesh,
        out_type=jax.ShapeDtypeStruct((NUM_LAYERS, n, d), jnp.float32),
        scratch_types=[
            pltpu.VMEM((tpw,), jnp.int32),
            pltpu.VMEM((chunk, d), jnp.float32),
            pltpu.SemaphoreType.DMA,
        ],
    )
    def k(idx_hbm, w0_hbm, w1_hbm, w2_hbm, out_hbm, idx_v, buf, gsem):
        wid = lax.axis_index("s") * nc + lax.axis_index("c")
        base = wid * tpw
        pltpu.sync_copy(idx_hbm.at[pl.ds(base, tpw)], idx_v)
        for t, w in enumerate((w0_hbm, w1_hbm, w2_hbm)):
            def body(g, carry, w=w, t=t):
                off = pl.multiple_of(g * chunk, chunk)
                pltpu.async_copy(w.at[idx_v.at[pl.ds(off, chunk)]], buf, gsem).wait()
                for r in range(NUM_LAYERS // NUM_TABLES):
                    pltpu.sync_copy(
                        buf, out_hbm.at[t + NUM_TABLES * r, pl.ds(base + off, chunk), :]
                    )
                return carry
            lax.fori_loop(0, nchunk, body, 0)

    return k(idx, w0, w1, w2)


def kernel(input_seq, W0, W1, W2):
    b, s = input_seq.shape
    _, d = W0.shape
    idx = input_seq.reshape(b * s)
    out = _sc_lookup(idx, W0, W1, W2)
    return out.reshape(NUM_LAYERS, b, s, d)


# speedup vs baseline: 1.6504x; 1.0045x over previous
import functools

import jax
import jax.numpy as jnp
from jax import lax
from jax.experimental import pallas as pl
from jax.experimental.pallas import tpu as pltpu
from jax.experimental.pallas import tpu_sc as plsc

NUM_LAYERS = 12
NUM_TABLES = 3


def _sc_lookup(idx, w0, w1, w2):
    (n,) = idx.shape
    v, d = w0.shape

    info = plsc.get_sparse_core_info()
    nc, ns = info.num_cores, info.num_subcores
    nw = nc * ns
    tpw = n // nw
    chunk = 32
    nchunk = tpw // chunk

    mesh = plsc.VectorSubcoreMesh(core_axis_name="c", subcore_axis_name="s")

    reps = NUM_LAYERS // NUM_TABLES
    nsteps = NUM_TABLES * nchunk

    @functools.partial(
        pl.kernel,
        mesh=mesh,
        out_type=jax.ShapeDtypeStruct((NUM_LAYERS, n, d), jnp.float32),
        scratch_types=[
            pltpu.VMEM((tpw,), jnp.int32),
            pltpu.VMEM((chunk, d), jnp.float32),
            pltpu.VMEM((chunk, d), jnp.float32),
            pltpu.SemaphoreType.DMA,
            pltpu.SemaphoreType.DMA,
            pltpu.SemaphoreType.DMA,
            pltpu.SemaphoreType.DMA,
        ],
    )
    def k(idx_hbm, w0_hbm, w1_hbm, w2_hbm, out_hbm, idx_v, buf0, buf1,
          gsem0, gsem1, ssem0, ssem1):
        wid = lax.axis_index("s") * nc + lax.axis_index("c")
        base = wid * tpw
        pltpu.sync_copy(idx_hbm.at[pl.ds(base, tpw)], idx_v)
        tables = (w0_hbm, w1_hbm, w2_hbm)
        bufs = (buf0, buf1)
        gsems = (gsem0, gsem1)
        ssems = (ssem0, ssem1)

        def gather(i):
            t, g = divmod(i, nchunk)
            s = i % 2
            return pltpu.async_copy(
                tables[t].at[idx_v.at[pl.ds(g * chunk, chunk)]], bufs[s], gsems[s]
            )

        def stores(i):
            t, g = divmod(i, nchunk)
            s = i % 2
            return [
                pltpu.async_copy(
                    bufs[s],
                    out_hbm.at[t + NUM_TABLES * r, pl.ds(base + g * chunk, chunk), :],
                    ssems[s],
                )
                for r in range(reps)
            ]

        pending_g = gather(0)
        pending_s = [None, None]
        for i in range(nsteps):
            if i + 1 < nsteps:
                if pending_s[(i + 1) % 2] is not None:
                    for c in pending_s[(i + 1) % 2]:
                        c.wait()
                next_g = gather(i + 1)
            pending_g.wait()
            pending_s[i % 2] = stores(i)
            if i + 1 < nsteps:
                pending_g = next_g
        for s in range(2):
            if pending_s[s] is not None:
                for c in pending_s[s]:
                    c.wait()

    return k(idx, w0, w1, w2)


def kernel(input_seq, W0, W1, W2):
    b, s = input_seq.shape
    _, d = W0.shape
    idx = input_seq.reshape(b * s)
    out = _sc_lookup(idx, W0, W1, W2)
    return out.reshape(NUM_LAYERS, b, s, d)
